# Initial kernel scaffold; baseline (speedup 1.0000x reference)
import jax
import jax.numpy as jnp
from jax.experimental import pallas as pl


def kernel(testTag, interaction, num_feat, emb_tag, emb_int, W_cat, b_cat, g1, beta1, W_num, b_num, g2, beta2, g3, beta3):
    raise NotImplementedError("write your pallas kernel here")


# speedup vs baseline: 2.2313x; 2.2313x over previous
import functools

import jax
import jax.numpy as jnp
from jax.experimental import pallas as pl
from jax.experimental.pallas import tpu as pltpu

B, L = 4096, 20
T = B * L
HIDDEN = 768
INTD = HIDDEN // 3
H2 = HIDDEN // 2
EPS = 1e-6
TB = 2048
GRID = T // TB


def _ln_rows(x, g, b):
    m = jnp.mean(x, axis=-1, keepdims=True)
    v = jnp.mean((x - m) ** 2, axis=-1, keepdims=True)
    return (x - m) * jax.lax.rsqrt(v + EPS) * g + b


def _body(tag_ref, inter_ref, num_ref, emb_tag_ref, emb_int_ref, W_cat_ref,
          b_cat_ref, g1_ref, beta1_ref, W_num_ref, b_num_ref, g2_ref,
          beta2_ref, g3_ref, beta3_ref, out_ref, tt_scratch, ti_scratch):
    @pl.when(pl.program_id(0) == 0)
    def _():
        tt_scratch[...] = jnp.dot(emb_tag_ref[...], W_cat_ref[:INTD, :],
                                  preferred_element_type=jnp.float32)
        ti_scratch[...] = jnp.dot(emb_int_ref[...], W_cat_ref[INTD:, :],
                                  preferred_element_type=jnp.float32)

    tag = tag_ref[...]
    inter = inter_ref[...]
    n = num_ref[...]

    ids = jax.lax.broadcasted_iota(jnp.int32, (TB, 16), 1)
    oh_t = (tag[:, None] == ids).astype(jnp.float32)
    oh_i = (inter[:, None] == ids).astype(jnp.float32)
    catp = (jnp.dot(oh_t, tt_scratch[...], preferred_element_type=jnp.float32)
            + jnp.dot(oh_i, ti_scratch[...], preferred_element_type=jnp.float32)
            + b_cat_ref[...])
    cat = _ln_rows(catp, g1_ref[...], beta1_ref[...])

    nump = n[:, None] * W_num_ref[0, :] + b_num_ref[...]
    num = _ln_rows(nump, g2_ref[...], beta2_ref[...])

    x = jnp.concatenate([cat, num], axis=-1)
    out_ref[...] = _ln_rows(x, g3_ref[...], beta3_ref[...])


@functools.partial(jax.jit, static_argnums=())
def kernel(testTag, interaction, num_feat, emb_tag, emb_int, W_cat, b_cat,
           g1, beta1, W_num, b_num, g2, beta2, g3, beta3):
    tag = testTag.reshape(T)
    inter = interaction.reshape(T)
    n = num_feat.reshape(T)
    emb_tag16 = jnp.zeros((16, INTD), jnp.float32).at[:10].set(emb_tag)
    emb_int16 = jnp.zeros((16, INTD), jnp.float32).at[:10].set(emb_int)

    rep = lambda shape: pl.BlockSpec(shape, lambda i: (0,) * len(shape))
    out = pl.pallas_call(
        _body,
        grid=(GRID,),
        in_specs=[
            pl.BlockSpec((TB,), lambda i: (i,)),
            pl.BlockSpec((TB,), lambda i: (i,)),
            pl.BlockSpec((TB,), lambda i: (i,)),
            rep((16, INTD)),
            rep((16, INTD)),
            rep((2 * INTD, H2)),
            rep((H2,)),
            rep((H2,)),
            rep((H2,)),
            rep((1, H2)),
            rep((H2,)),
            rep((H2,)),
            rep((H2,)),
            rep((HIDDEN,)),
            rep((HIDDEN,)),
        ],
        out_specs=pl.BlockSpec((TB, HIDDEN), lambda i: (i, 0)),
        out_shape=jax.ShapeDtypeStruct((T, HIDDEN), jnp.float32),
        scratch_shapes=[
            pltpu.VMEM((16, H2), jnp.float32),
            pltpu.VMEM((16, H2), jnp.float32),
        ],
    )(tag, inter, n, emb_tag16, emb_int16, W_cat, b_cat, g1, beta1,
      W_num, b_num, g2, beta2, g3, beta3)
    return out.reshape(B, L, HIDDEN)
